# baseline (speedup 1.0000x reference)
import functools

import jax
import jax.numpy as jnp
from jax import lax
from jax.experimental import pallas as pl
from jax.experimental.pallas import tpu as pltpu
from jax.experimental.pallas import tpu_sc as plsc

D = 64
SCALE = 8.0
HIST = 200
TAIL = HIST - 128


def _split_body(tok_ref, t0_ref, t1_ref):
    x = tok_ref[...]
    t0_ref[...] = x[:, :128]
    t1_ref[...] = x[:, HIST - 128:HIST]


def _gather_body(t0_hbm, t1_hbm, table_hbm, out_hbm, idx0_v, idx1_v, rows_v,
                 sem, *, rows_per_w, nc):
    wid = lax.axis_index("s") * nc + lax.axis_index("c")
    b0 = pl.multiple_of(wid * rows_per_w, 8)
    pltpu.sync_copy(t0_hbm.at[pl.ds(b0, rows_per_w)], idx0_v)
    pltpu.sync_copy(t1_hbm.at[pl.ds(b0, rows_per_w)], idx1_v)

    def row_body(j, carry):
        cp0 = pltpu.async_copy(
            table_hbm.at[idx0_v.at[j]], rows_v.at[pl.ds(0, 128)], sem)
        cp1 = pltpu.async_copy(
            table_hbm.at[idx1_v.at[j, pl.ds(128 - TAIL, TAIL)]],
            rows_v.at[pl.ds(128, TAIL)], sem)
        cp0.wait()
        cp1.wait()
        off = pl.multiple_of((b0 + j) * HIST, 8)
        pltpu.sync_copy(rows_v, out_hbm.at[pl.ds(off, HIST)])
        return carry

    lax.fori_loop(0, rows_per_w, row_body, 0)


def _finish_body(flat_ref, out_ref, *, blk_b):
    for i in range(blk_b):
        out_ref[i] = flat_ref[pl.ds(i * HIST, HIST)] * SCALE


def kernel(tokens, table):
    batch, hist = tokens.shape
    assert hist == HIST
    info = plsc.get_sparse_core_info()
    nc, ns = info.num_cores, info.num_subcores
    nw = nc * ns
    rows_per_w = batch // nw

    tok = tokens.astype(jnp.int32)

    split_blk = 512
    t0, t1 = pl.pallas_call(
        _split_body,
        grid=(batch // split_blk,),
        in_specs=[pl.BlockSpec((split_blk, HIST), lambda i: (i, 0))],
        out_specs=[
            pl.BlockSpec((split_blk, 128), lambda i: (i, 0)),
            pl.BlockSpec((split_blk, 128), lambda i: (i, 0)),
        ],
        out_shape=[
            jax.ShapeDtypeStruct((batch, 128), jnp.int32),
            jax.ShapeDtypeStruct((batch, 128), jnp.int32),
        ],
    )(tok)

    mesh = plsc.VectorSubcoreMesh(core_axis_name="c", subcore_axis_name="s")
    flat = pl.kernel(
        functools.partial(_gather_body, rows_per_w=rows_per_w, nc=nc),
        mesh=mesh,
        out_type=jax.ShapeDtypeStruct((batch * HIST, D), jnp.float32),
        scratch_types=[
            pltpu.VMEM((rows_per_w, 128), jnp.int32),
            pltpu.VMEM((rows_per_w, 128), jnp.int32),
            pltpu.VMEM((HIST, D), jnp.float32),
            pltpu.SemaphoreType.DMA,
        ],
        compiler_params=pltpu.CompilerParams(use_tc_tiling_on_sc=False),
    )(t0, t1, table)

    blk_b = 64
    out = pl.pallas_call(
        functools.partial(_finish_body, blk_b=blk_b),
        grid=(batch // blk_b,),
        in_specs=[pl.BlockSpec((blk_b * HIST, D), lambda i: (i, 0))],
        out_specs=pl.BlockSpec((blk_b, HIST, D), lambda i: (i, 0, 0)),
        out_shape=jax.ShapeDtypeStruct((batch, HIST, D), jnp.float32),
    )(flat)
    return out
